# baseline (speedup 1.0000x reference)
import functools

import jax
import jax.numpy as jnp
from jax import lax
from jax.experimental import pallas as pl
from jax.experimental.pallas import tpu as pltpu
from jax.experimental.pallas import tpu_sc as plsc

CU = 10.0
ALPHA = 5.0 / CU
NC = 2
NS = 16
NW = NC * NS


def _round_up(x, m):
    return (x + m - 1) // m * m


def _group_pos_kernel(pos_ref, z_ref, lab_ref, gp_ref, acc_ref):
    i = pl.program_id(0)

    @pl.when(i == 0)
    def _():
        acc_ref[...] = jnp.zeros_like(acc_ref)

    zf = z_ref[0, 0, :].astype(jnp.float32)
    lab = lab_ref[0, 0, :]
    posb = pos_ref[0]
    w4 = jnp.concatenate([posb * zf[:, None], zf[:, None]], axis=1)
    oh_t = (lax.broadcasted_iota(jnp.int32, (512, lab.shape[0]), 0)
            == lab[None, :]).astype(jnp.float32)
    acc_ref[...] += jnp.dot(oh_t, w4, preferred_element_type=jnp.float32)

    @pl.when(i == pl.num_programs(0) - 1)
    def _():
        acc = acc_ref[...]
        den = acc[:, 3:4]
        gp = acc[:, :3] / jnp.where(den > 0, den, 1.0)
        gp_ref[...] = jnp.concatenate(
            [gp, jnp.zeros((512, 1), jnp.float32)], axis=1)


def _sc_gather_body(n_edges, n_bip, src_hbm, dst_hbm, nid_hbm, gid_hbm,
                    z_hbm, pos_hbm, gp_hbm,
                    ew2_out, bw2_out, zsrc_out,
                    pos_v, z_v, gp_v, src_v, dst_v, zsrc_v, ew2_v,
                    nid_v, gid_v, bw2_v):
    ept = src_v.shape[0]
    bpt = nid_v.shape[0]
    wid = lax.axis_index("c") * NS + lax.axis_index("s")
    base = wid * ept
    bbase = wid * bpt

    pltpu.sync_copy(pos_hbm, pos_v)
    pltpu.sync_copy(z_hbm, z_v)
    pltpu.sync_copy(gp_hbm, gp_v)
    pltpu.sync_copy(src_hbm.at[pl.ds(base, ept)], src_v)
    pltpu.sync_copy(dst_hbm.at[pl.ds(base, ept)], dst_v)
    pltpu.sync_copy(nid_hbm.at[pl.ds(bbase, bpt)], nid_v)
    pltpu.sync_copy(gid_hbm.at[pl.ds(bbase, bpt)], gid_v)

    lane = lax.iota(jnp.int32, 16)

    def edge_body(k, _):
        s4 = src_v[pl.ds(k * 16, 16)] * 4
        d4 = dst_v[pl.ds(k * 16, 16)] * 4
        dx = plsc.load_gather(pos_v, [s4]) - plsc.load_gather(pos_v, [d4])
        dy = (plsc.load_gather(pos_v, [s4 + 1])
              - plsc.load_gather(pos_v, [d4 + 1]))
        dz = (plsc.load_gather(pos_v, [s4 + 2])
              - plsc.load_gather(pos_v, [d4 + 2]))
        ew2 = dx * dx + dy * dy + dz * dz
        ge = base + k * 16 + lane
        ok = (ge < n_edges) & (s4 != d4)
        ew2_v[pl.ds(k * 16, 16)] = jnp.where(ok, ew2, -1.0)
        zsrc_v[pl.ds(k * 16, 16)] = plsc.load_gather(
            z_v, [src_v[pl.ds(k * 16, 16)]])
        return 0

    lax.fori_loop(0, ept // 16, edge_body, 0)

    def bip_body(k, _):
        n4 = nid_v[pl.ds(k * 16, 16)] * 4
        g4 = gid_v[pl.ds(k * 16, 16)] * 4
        dx = plsc.load_gather(pos_v, [n4]) - plsc.load_gather(gp_v, [g4])
        dy = (plsc.load_gather(pos_v, [n4 + 1])
              - plsc.load_gather(gp_v, [g4 + 1]))
        dz = (plsc.load_gather(pos_v, [n4 + 2])
              - plsc.load_gather(gp_v, [g4 + 2]))
        bw2 = dx * dx + dy * dy + dz * dz
        ge = bbase + k * 16 + lane
        bw2_v[pl.ds(k * 16, 16)] = jnp.where(ge < n_bip, bw2, 1e9)
        return 0

    lax.fori_loop(0, bpt // 16, bip_body, 0)

    pltpu.sync_copy(ew2_v, ew2_out.at[pl.ds(base, ept)])
    pltpu.sync_copy(bw2_v, bw2_out.at[pl.ds(bbase, bpt)])
    pltpu.sync_copy(zsrc_v, zsrc_out.at[pl.ds(base, ept)])


def _edge_kernel(ew2_ref, zsrc_ref, emb_ref, means_ref, betas_ref, wd_ref,
                 bd_ref, msg_ref, eaug_ref):
    ew2m = ew2_ref[0, 0, :]
    zsrc = zsrc_ref[0, 0, :]
    oh = (lax.broadcasted_iota(jnp.int32, (zsrc.shape[0], 128), 1)
          == zsrc[:, None]).astype(jnp.float32)
    xsrc = jnp.dot(oh, emb_ref[...], preferred_element_type=jnp.float32)
    kf = (ew2m >= 0).astype(jnp.float32)
    ew = jnp.sqrt(jnp.maximum(ew2m, 0.0))
    cc = jnp.where(ew < CU, 0.5 * (jnp.cos(jnp.pi * ew / CU) + 1.0), 0.0)
    u = jnp.exp(-ALPHA * ew)
    t = u[:, None] - means_ref[0, :][None, :]
    ea = cc[:, None] * jnp.exp(-betas_ref[0, :][None, :] * t * t)
    cvec = cc * kf
    wmsg = (jnp.dot(ea, wd_ref[...], preferred_element_type=jnp.float32)
            + bd_ref[0, :][None, :]) * cvec[:, None]
    msg_ref[...] = xsrc * wmsg
    col = lax.broadcasted_iota(jnp.int32, ea.shape, 1)
    part = jnp.where(col < 50, ea * kf[:, None],
                     jnp.where(col == 50, kf[:, None], 0.0))
    eaug_ref[...] = jnp.concatenate(
        [part, jnp.zeros_like(part)], axis=1)


def _bip_kernel(bw2_ref, means_ref, betas_ref, out_ref):
    bw2 = bw2_ref[0, 0, :]
    bw = jnp.sqrt(bw2)
    bf = (bw <= CU).astype(jnp.float32)
    cc = jnp.where(bw < CU, 0.5 * (jnp.cos(jnp.pi * bw / CU) + 1.0), 0.0)
    u = jnp.exp(-ALPHA * bw)
    t = u[:, None] - means_ref[0, :][None, :]
    ea = cc[:, None] * jnp.exp(-betas_ref[0, :][None, :] * t * t)
    col = lax.broadcasted_iota(jnp.int32, ea.shape, 1)
    part = jnp.where(col < 50, ea * bf[:, None],
                     jnp.where(col == 50, bf[:, None], 0.0))
    out_ref[...] = jnp.concatenate([part, jnp.zeros_like(part)], axis=1)


def _sc_scatter_body(n_rows, ept1, ept3, rows1_hbm, idx1_hbm, rows2_hbm,
                     rows3_hbm, idx3_hbm, out_hbm, acc, buf, idx_v,
                     sem_r, sem_s):
    c = lax.axis_index("c")
    s = lax.axis_index("s")
    wid = c * NS + s
    rows_per_tile = n_rows // NS
    lane = lax.iota(jnp.int32, 16)

    def phase(rows_hbm, idx_hbm, oi, ept, ch, stage_idx):
        def bfill(i, _):
            r = i // 8
            cc = (i % 8) * 16
            buf[0, r, pl.ds(cc, 16)] = jnp.zeros((16,), jnp.float32)
            return 0

        lax.fori_loop(0, 128 * 8, bfill, 0)

        def ifill(i, _):
            idx_v[pl.ds(i * 16, 16)] = s * rows_per_tile + i * 16 + lane
            return 0

        lax.fori_loop(0, rows_per_tile // 16, ifill, 0)

        def zcp(q, _):
            pltpu.sync_copy(buf.at[0, pl.ds(0, 128)],
                            acc.at[idx_v.at[pl.ds(q * 128, 128)]])
            return 0

        lax.fori_loop(0, rows_per_tile // 128, zcp, 0)
        plsc.subcore_barrier()
        if stage_idx:
            pltpu.sync_copy(idx_hbm.at[pl.ds(wid * ept, ept)],
                            idx_v.at[pl.ds(0, ept)])

        nk = ept // ch

        def slot(b):
            return buf.at[b] if ch == buf.shape[1] else buf.at[b,
                                                              pl.ds(0, ch)]

        def r_desc(k, b):
            return pltpu.make_async_copy(
                rows_hbm.at[pl.ds(wid * ept + k * ch, ch)], slot(b), sem_r)

        def s_desc(k, b):
            return pltpu.make_async_copy(
                slot(b), acc.at[idx_v.at[pl.ds(k * ch, ch)]], sem_s)

        def s_start(k, b):
            pltpu.async_copy(slot(b), acc.at[idx_v.at[pl.ds(k * ch, ch)]],
                             sem_s, add=True)

        r_desc(0, 0).start()

        def pair_body(j, _):
            k0 = 2 * j

            @pl.when(j >= 1)
            def _():
                s_desc(k0 - 1, 1).wait()

            r_desc(k0 + 1, 1).start()
            r_desc(k0, 0).wait()
            s_start(k0, 0)

            @pl.when(k0 + 2 < nk)
            def _():
                s_desc(k0, 0).wait()
                r_desc(k0 + 2, 0).start()

            r_desc(k0 + 1, 1).wait()
            s_start(k0 + 1, 1)
            return 0

        lax.fori_loop(0, nk // 2, pair_body, 0)
        s_desc(nk - 2, 0).wait()
        s_desc(nk - 1, 1).wait()
        plsc.subcore_barrier()
        pltpu.sync_copy(acc.at[pl.ds(s * rows_per_tile, rows_per_tile)],
                        out_hbm.at[c, oi,
                                   pl.ds(s * rows_per_tile, rows_per_tile)])
        plsc.subcore_barrier()

    phase(rows1_hbm, idx1_hbm, 0, ept1, 128, True)
    phase(rows2_hbm, idx1_hbm, 1, ept1, 128, False)
    phase(rows3_hbm, idx3_hbm, 2, ept3, 128, True)


def _final_kernel(parts_ref, z_ref, emb_ref, wct_ref, wcb_ref,
                  bc_ref, wm_ref, bm_ref, wrs_ref, wrb_ref, out_ref):
    agg = parts_ref[0, 0] + parts_ref[1, 0]
    sacc = parts_ref[0, 1] + parts_ref[1, 1]
    acc2 = parts_ref[0, 2] + parts_ref[1, 2]
    zb = z_ref[0, 0, :]
    oh = (lax.broadcasted_iota(jnp.int32, (zb.shape[0], 128), 1)
          == zb[:, None]).astype(jnp.float32)
    nx = jnp.dot(oh, emb_ref[...], preferred_element_type=jnp.float32)
    h = (jnp.dot(nx, wct_ref[...], preferred_element_type=jnp.float32)
         + jnp.dot(agg, wcb_ref[...], preferred_element_type=jnp.float32)
         + bc_ref[0, :][None, :])
    node_cat = (jnp.dot(h, wm_ref[...], preferred_element_type=jnp.float32)
                + bm_ref[0, :][None, :])
    out_s = jnp.dot(sacc, wrs_ref[...], preferred_element_type=jnp.float32)
    out_l = jnp.dot(acc2, wrb_ref[...], preferred_element_type=jnp.float32)
    out_ref[...] = node_cat + jnp.concatenate([out_s, out_l], axis=1)


def kernel(z, pos, labels, edge_index, interaction_graph, emb, ne_emb,
           ne_Wd, ne_bd, ne_Wc, ne_bc, means_s, betas_s, Wr_s, br_s,
           means_b, betas_b, Wr_b, br_b, Wm, bm):
    n = z.shape[0]
    n_edges = edge_index.shape[1]
    n_bip = interaction_graph.shape[1]
    f32 = jnp.float32

    npad = _round_up(n, 2048)
    epad = _round_up(n_edges, 8192)
    bpad = _round_up(n_bip, 8192)
    ept = epad // NW
    bpt = bpad // NW
    nb = 2000
    eblk = 2048

    srcp = jnp.pad(edge_index[0], (0, epad - n_edges))
    dstp = jnp.pad(edge_index[1], (0, epad - n_edges))
    nidp = jnp.pad(interaction_graph[0], (0, bpad - n_bip))
    gidp = jnp.pad(interaction_graph[1], (0, bpad - n_bip))
    pos4 = jnp.pad(pos, ((0, 0), (0, 1)))
    z3d = z.reshape(n // nb, 1, nb)
    lab3d = labels.reshape(n // nb, 1, nb)
    pos3d = pos.reshape(n // nb, nb, 3)
    means_sp = jnp.pad(means_s, (0, 14)).reshape(1, 64)
    betas_sp = jnp.pad(betas_s, (0, 14)).reshape(1, 64)
    means_bp = jnp.pad(means_b, (0, 14)).reshape(1, 64)
    betas_bp = jnp.pad(betas_b, (0, 14)).reshape(1, 64)
    wdp = jnp.pad(ne_Wd, ((0, 14), (0, 0)))
    embp = jnp.pad(emb, ((0, 128 - emb.shape[0]), (0, 0)))
    ne_embp = jnp.pad(ne_emb, ((0, 128 - ne_emb.shape[0]), (0, 0)))
    wrs_aug = jnp.concatenate(
        [Wr_s, br_s[None, :], jnp.zeros((77, 128), f32)], axis=0)
    wrb_aug = jnp.concatenate(
        [Wr_b, br_b[None, :], jnp.zeros((77, 128), f32)], axis=0)

    gp512 = pl.pallas_call(
        _group_pos_kernel,
        grid=(n // nb,),
        in_specs=[
            pl.BlockSpec((1, nb, 3), lambda i: (i, 0, 0)),
            pl.BlockSpec((1, 1, nb), lambda i: (i, 0, 0)),
            pl.BlockSpec((1, 1, nb), lambda i: (i, 0, 0)),
        ],
        out_specs=pl.BlockSpec((512, 4), lambda i: (0, 0)),
        out_shape=jax.ShapeDtypeStruct((512, 4), f32),
        scratch_shapes=[pltpu.VMEM((512, 4), f32)],
        name="group_pos",
    )(pos3d, z3d, lab3d)

    mesh = plsc.VectorSubcoreMesh(core_axis_name="c", subcore_axis_name="s")
    ew2m, bw2m, zsrc = pl.kernel(
        functools.partial(_sc_gather_body, n_edges, n_bip),
        out_type=[
            jax.ShapeDtypeStruct((epad,), f32),
            jax.ShapeDtypeStruct((bpad,), f32),
            jax.ShapeDtypeStruct((epad,), jnp.int32),
        ],
        mesh=mesh,
        scratch_types=[
            pltpu.VMEM((n * 4,), f32),
            pltpu.VMEM((n,), jnp.int32),
            pltpu.VMEM((2048,), f32),
            pltpu.VMEM((ept,), jnp.int32),
            pltpu.VMEM((ept,), jnp.int32),
            pltpu.VMEM((ept,), jnp.int32),
            pltpu.VMEM((ept,), f32),
            pltpu.VMEM((bpt,), jnp.int32),
            pltpu.VMEM((bpt,), jnp.int32),
            pltpu.VMEM((bpt,), f32),
        ],
        compiler_params=pltpu.CompilerParams(needs_layout_passes=False),
        name="sc_gather",
    )(srcp, dstp, nidp, gidp, z, pos4.reshape(-1), gp512.reshape(-1))

    msg, eaug = pl.pallas_call(
        _edge_kernel,
        grid=(epad // eblk,),
        in_specs=[
            pl.BlockSpec((1, 1, eblk), lambda i: (i, 0, 0)),
            pl.BlockSpec((1, 1, eblk), lambda i: (i, 0, 0)),
            pl.BlockSpec((128, 128), lambda i: (0, 0)),
            pl.BlockSpec((1, 64), lambda i: (0, 0)),
            pl.BlockSpec((1, 64), lambda i: (0, 0)),
            pl.BlockSpec((64, 128), lambda i: (0, 0)),
            pl.BlockSpec((1, 128), lambda i: (0, 0)),
        ],
        out_specs=[
            pl.BlockSpec((eblk, 128), lambda i: (i, 0)),
            pl.BlockSpec((eblk, 128), lambda i: (i, 0)),
        ],
        out_shape=[
            jax.ShapeDtypeStruct((epad, 128), f32),
            jax.ShapeDtypeStruct((epad, 128), f32),
        ],
        name="edge_rows",
    )(ew2m.reshape(epad // eblk, 1, eblk),
      zsrc.reshape(epad // eblk, 1, eblk), ne_embp, means_sp, betas_sp,
      wdp, ne_bd.reshape(1, 128))

    baug = pl.pallas_call(
        _bip_kernel,
        grid=(bpad // eblk,),
        in_specs=[
            pl.BlockSpec((1, 1, eblk), lambda i: (i, 0, 0)),
            pl.BlockSpec((1, 64), lambda i: (0, 0)),
            pl.BlockSpec((1, 64), lambda i: (0, 0)),
        ],
        out_specs=pl.BlockSpec((eblk, 128), lambda i: (i, 0)),
        out_shape=jax.ShapeDtypeStruct((bpad, 128), f32),
        name="bip_rows",
    )(bw2m.reshape(bpad // eblk, 1, eblk), means_bp, betas_bp)

    parts = pl.kernel(
        functools.partial(_sc_scatter_body, npad, ept, bpt),
        out_type=jax.ShapeDtypeStruct((NC, 3, npad, 128), f32),
        mesh=mesh,
        scratch_types=[
            pltpu.VMEM_SHARED((npad, 128), f32),
            pltpu.VMEM((2, 128, 128), f32),
            pltpu.VMEM((ept,), jnp.int32),
            pltpu.SemaphoreType.DMA,
            pltpu.SemaphoreType.DMA,
        ],
        name="sc_scatter",
    )(msg, dstp, eaug, baug, nidp)

    nbd = 2048
    z3d_d = jnp.pad(z, (0, npad - n)).reshape(npad // nbd, 1, nbd)
    out = pl.pallas_call(
        _final_kernel,
        grid=(npad // nbd,),
        in_specs=[
            pl.BlockSpec((NC, 3, nbd, 128), lambda i: (0, 0, i, 0)),
            pl.BlockSpec((1, 1, nbd), lambda i: (i, 0, 0)),
            pl.BlockSpec((128, 128), lambda i: (0, 0)),
            pl.BlockSpec((128, 128), lambda i: (0, 0)),
            pl.BlockSpec((128, 128), lambda i: (0, 0)),
            pl.BlockSpec((1, 128), lambda i: (0, 0)),
            pl.BlockSpec((128, 256), lambda i: (0, 0)),
            pl.BlockSpec((1, 256), lambda i: (0, 0)),
            pl.BlockSpec((128, 128), lambda i: (0, 0)),
            pl.BlockSpec((128, 128), lambda i: (0, 0)),
        ],
        out_specs=pl.BlockSpec((nbd, 256), lambda i: (i, 0)),
        out_shape=jax.ShapeDtypeStruct((npad, 256), f32),
        name="node_final",
    )(parts, z3d_d, embp, ne_Wc[:128], ne_Wc[128:],
      ne_bc.reshape(1, 128), Wm, bm.reshape(1, 256), wrs_aug, wrb_aug)

    return out[:n]


# speedup vs baseline: 1.0483x; 1.0483x over previous
import functools

import jax
import jax.numpy as jnp
from jax import lax
from jax.experimental import pallas as pl
from jax.experimental.pallas import tpu as pltpu
from jax.experimental.pallas import tpu_sc as plsc

CU = 10.0
ALPHA = 5.0 / CU
NC = 2
NS = 16
NW = NC * NS


def _round_up(x, m):
    return (x + m - 1) // m * m


def _group_pos_kernel(pos_ref, z_ref, lab_ref, gp_ref, acc_ref):
    i = pl.program_id(0)

    @pl.when(i == 0)
    def _():
        acc_ref[...] = jnp.zeros_like(acc_ref)

    zf = z_ref[0, 0, :].astype(jnp.float32)
    lab = lab_ref[0, 0, :]
    posb = pos_ref[0]
    w4 = jnp.concatenate([posb * zf[:, None], zf[:, None]], axis=1)
    oh_t = (lax.broadcasted_iota(jnp.int32, (512, lab.shape[0]), 0)
            == lab[None, :]).astype(jnp.float32)
    acc_ref[...] += jnp.dot(oh_t, w4, preferred_element_type=jnp.float32)

    @pl.when(i == pl.num_programs(0) - 1)
    def _():
        acc = acc_ref[...]
        den = acc[:, 3:4]
        gp = acc[:, :3] / jnp.where(den > 0, den, 1.0)
        gp_ref[...] = jnp.concatenate(
            [gp, jnp.zeros((512, 1), jnp.float32)], axis=1)


def _sc_gather_body(n_edges, n_bip, src_hbm, dst_hbm, nid_hbm, gid_hbm,
                    z_hbm, pos_hbm, gp_hbm,
                    ew2_out, bw2_out, zsrc_out,
                    pos_v, z_v, gp_v, src_v, dst_v, zsrc_v, ew2_v,
                    nid_v, gid_v, bw2_v):
    ept = src_v.shape[0]
    bpt = nid_v.shape[0]
    wid = lax.axis_index("c") * NS + lax.axis_index("s")
    base = wid * ept
    bbase = wid * bpt

    pltpu.sync_copy(pos_hbm, pos_v)
    pltpu.sync_copy(z_hbm, z_v)
    pltpu.sync_copy(gp_hbm, gp_v)
    pltpu.sync_copy(src_hbm.at[pl.ds(base, ept)], src_v)
    pltpu.sync_copy(dst_hbm.at[pl.ds(base, ept)], dst_v)
    pltpu.sync_copy(nid_hbm.at[pl.ds(bbase, bpt)], nid_v)
    pltpu.sync_copy(gid_hbm.at[pl.ds(bbase, bpt)], gid_v)

    lane = lax.iota(jnp.int32, 16)

    def edge_body(k, _):
        s4 = src_v[pl.ds(k * 16, 16)] * 4
        d4 = dst_v[pl.ds(k * 16, 16)] * 4
        dx = plsc.load_gather(pos_v, [s4]) - plsc.load_gather(pos_v, [d4])
        dy = (plsc.load_gather(pos_v, [s4 + 1])
              - plsc.load_gather(pos_v, [d4 + 1]))
        dz = (plsc.load_gather(pos_v, [s4 + 2])
              - plsc.load_gather(pos_v, [d4 + 2]))
        ew2 = dx * dx + dy * dy + dz * dz
        ge = base + k * 16 + lane
        ok = (ge < n_edges) & (s4 != d4)
        ew2_v[pl.ds(k * 16, 16)] = jnp.where(ok, ew2, -1.0)
        zsrc_v[pl.ds(k * 16, 16)] = plsc.load_gather(
            z_v, [src_v[pl.ds(k * 16, 16)]])
        return 0

    lax.fori_loop(0, ept // 16, edge_body, 0)

    def bip_body(k, _):
        n4 = nid_v[pl.ds(k * 16, 16)] * 4
        g4 = gid_v[pl.ds(k * 16, 16)] * 4
        dx = plsc.load_gather(pos_v, [n4]) - plsc.load_gather(gp_v, [g4])
        dy = (plsc.load_gather(pos_v, [n4 + 1])
              - plsc.load_gather(gp_v, [g4 + 1]))
        dz = (plsc.load_gather(pos_v, [n4 + 2])
              - plsc.load_gather(gp_v, [g4 + 2]))
        bw2 = dx * dx + dy * dy + dz * dz
        ge = bbase + k * 16 + lane
        bw2_v[pl.ds(k * 16, 16)] = jnp.where(ge < n_bip, bw2, 1e9)
        return 0

    lax.fori_loop(0, bpt // 16, bip_body, 0)

    pltpu.sync_copy(ew2_v, ew2_out.at[pl.ds(base, ept)])
    pltpu.sync_copy(bw2_v, bw2_out.at[pl.ds(bbase, bpt)])
    pltpu.sync_copy(zsrc_v, zsrc_out.at[pl.ds(base, ept)])


def _edge_kernel(ew2_ref, zsrc_ref, emb_ref, means_ref, betas_ref, wd_ref,
                 bd_ref, msg_ref, eaug_ref):
    ew2m = ew2_ref[0, 0, :]
    zsrc = zsrc_ref[0, 0, :]
    oh = (lax.broadcasted_iota(jnp.int32, (zsrc.shape[0], 128), 1)
          == zsrc[:, None]).astype(jnp.float32)
    xsrc = jnp.dot(oh, emb_ref[...], preferred_element_type=jnp.float32)
    kf = (ew2m >= 0).astype(jnp.float32)
    ew = jnp.sqrt(jnp.maximum(ew2m, 0.0))
    cc = jnp.where(ew < CU, 0.5 * (jnp.cos(jnp.pi * ew / CU) + 1.0), 0.0)
    u = jnp.exp(-ALPHA * ew)
    t = u[:, None] - means_ref[0, :][None, :]
    ea = cc[:, None] * jnp.exp(-betas_ref[0, :][None, :] * t * t)
    cvec = cc * kf
    wmsg = (jnp.dot(ea, wd_ref[...], preferred_element_type=jnp.float32)
            + bd_ref[0, :][None, :]) * cvec[:, None]
    msg_ref[...] = xsrc * wmsg
    col = lax.broadcasted_iota(jnp.int32, ea.shape, 1)
    part = jnp.where(col < 50, ea * kf[:, None],
                     jnp.where(col == 50, kf[:, None], 0.0))
    eaug_ref[...] = jnp.concatenate(
        [part, jnp.zeros_like(part)], axis=1)


def _bip_kernel(bw2_ref, means_ref, betas_ref, out_ref):
    bw2 = bw2_ref[0, 0, :]
    bw = jnp.sqrt(bw2)
    bf = (bw <= CU).astype(jnp.float32)
    cc = jnp.where(bw < CU, 0.5 * (jnp.cos(jnp.pi * bw / CU) + 1.0), 0.0)
    u = jnp.exp(-ALPHA * bw)
    t = u[:, None] - means_ref[0, :][None, :]
    ea = cc[:, None] * jnp.exp(-betas_ref[0, :][None, :] * t * t)
    col = lax.broadcasted_iota(jnp.int32, ea.shape, 1)
    part = jnp.where(col < 50, ea * bf[:, None],
                     jnp.where(col == 50, bf[:, None], 0.0))
    out_ref[...] = jnp.concatenate([part, jnp.zeros_like(part)], axis=1)


def _sc_scatter_body(n_rows, rows_hbm, idx_hbm, out_hbm, acc, buf, idx_v,
                     sem_r, sem_s):
    ch = buf.shape[1]
    ept = idx_v.shape[0]
    c = lax.axis_index("c")
    s = lax.axis_index("s")
    wid = c * NS + s
    rows_per_tile = n_rows // NS
    lane = lax.iota(jnp.int32, 16)

    def bfill(i, _):
        r = i // 8
        cc = (i % 8) * 16
        buf[0, r, pl.ds(cc, 16)] = jnp.zeros((16,), jnp.float32)
        return 0

    lax.fori_loop(0, 128 * 8, bfill, 0)

    def ifill(i, _):
        idx_v[pl.ds(i * 16, 16)] = s * rows_per_tile + i * 16 + lane
        return 0

    lax.fori_loop(0, rows_per_tile // 16, ifill, 0)

    def zcp(q, _):
        pltpu.sync_copy(buf.at[0],
                        acc.at[idx_v.at[pl.ds(q * 128, 128)]])
        return 0

    lax.fori_loop(0, rows_per_tile // 128, zcp, 0)
    plsc.subcore_barrier()
    pltpu.sync_copy(idx_hbm.at[pl.ds(wid * ept, ept)], idx_v)

    nk = ept // ch

    def r_desc(k, b):
        return pltpu.make_async_copy(
            rows_hbm.at[pl.ds(wid * ept + k * ch, ch)], buf.at[b], sem_r)

    def s_desc(k, b):
        return pltpu.make_async_copy(
            buf.at[b], acc.at[idx_v.at[pl.ds(k * ch, ch)]], sem_s)

    def s_start(k, b):
        pltpu.async_copy(buf.at[b], acc.at[idx_v.at[pl.ds(k * ch, ch)]],
                         sem_s, add=True)

    r_desc(0, 0).start()

    def pair_body(j, _):
        k0 = 2 * j

        @pl.when(j >= 1)
        def _():
            s_desc(k0 - 1, 1).wait()

        r_desc(k0 + 1, 1).start()
        r_desc(k0, 0).wait()
        s_start(k0, 0)

        @pl.when(k0 + 2 < nk)
        def _():
            s_desc(k0, 0).wait()
            r_desc(k0 + 2, 0).start()

        r_desc(k0 + 1, 1).wait()
        s_start(k0 + 1, 1)
        return 0

    lax.fori_loop(0, nk // 2, pair_body, 0)
    s_desc(nk - 2, 0).wait()
    s_desc(nk - 1, 1).wait()
    plsc.subcore_barrier()
    pltpu.sync_copy(acc.at[pl.ds(s * rows_per_tile, rows_per_tile)],
                    out_hbm.at[c, pl.ds(s * rows_per_tile, rows_per_tile)])


def _final_kernel(pm_ref, ps_ref, pl_ref, z_ref, emb_ref, wct_ref, wcb_ref,
                  bc_ref, wm_ref, bm_ref, wrs_ref, wrb_ref, out_ref):
    agg = pm_ref[0] + pm_ref[1]
    sacc = ps_ref[0] + ps_ref[1]
    acc2 = pl_ref[0] + pl_ref[1]
    zb = z_ref[0, 0, :]
    oh = (lax.broadcasted_iota(jnp.int32, (zb.shape[0], 128), 1)
          == zb[:, None]).astype(jnp.float32)
    nx = jnp.dot(oh, emb_ref[...], preferred_element_type=jnp.float32)
    h = (jnp.dot(nx, wct_ref[...], preferred_element_type=jnp.float32)
         + jnp.dot(agg, wcb_ref[...], preferred_element_type=jnp.float32)
         + bc_ref[0, :][None, :])
    node_cat = (jnp.dot(h, wm_ref[...], preferred_element_type=jnp.float32)
                + bm_ref[0, :][None, :])
    out_s = jnp.dot(sacc, wrs_ref[...], preferred_element_type=jnp.float32)
    out_l = jnp.dot(acc2, wrb_ref[...], preferred_element_type=jnp.float32)
    out_ref[...] = node_cat + jnp.concatenate([out_s, out_l], axis=1)


def kernel(z, pos, labels, edge_index, interaction_graph, emb, ne_emb,
           ne_Wd, ne_bd, ne_Wc, ne_bc, means_s, betas_s, Wr_s, br_s,
           means_b, betas_b, Wr_b, br_b, Wm, bm):
    n = z.shape[0]
    n_edges = edge_index.shape[1]
    n_bip = interaction_graph.shape[1]
    f32 = jnp.float32

    npad = _round_up(n, 2048)
    epad = _round_up(n_edges, 8192)
    bpad = _round_up(n_bip, 8192)
    ept = epad // NW
    bpt = bpad // NW
    nb = 2000
    eblk = 2048

    srcp = jnp.pad(edge_index[0], (0, epad - n_edges))
    dstp = jnp.pad(edge_index[1], (0, epad - n_edges))
    nidp = jnp.pad(interaction_graph[0], (0, bpad - n_bip))
    gidp = jnp.pad(interaction_graph[1], (0, bpad - n_bip))
    pos4 = jnp.pad(pos, ((0, 0), (0, 1)))
    z3d = z.reshape(n // nb, 1, nb)
    lab3d = labels.reshape(n // nb, 1, nb)
    pos3d = pos.reshape(n // nb, nb, 3)
    means_sp = jnp.pad(means_s, (0, 14)).reshape(1, 64)
    betas_sp = jnp.pad(betas_s, (0, 14)).reshape(1, 64)
    means_bp = jnp.pad(means_b, (0, 14)).reshape(1, 64)
    betas_bp = jnp.pad(betas_b, (0, 14)).reshape(1, 64)
    wdp = jnp.pad(ne_Wd, ((0, 14), (0, 0)))
    embp = jnp.pad(emb, ((0, 128 - emb.shape[0]), (0, 0)))
    ne_embp = jnp.pad(ne_emb, ((0, 128 - ne_emb.shape[0]), (0, 0)))
    wrs_aug = jnp.concatenate(
        [Wr_s, br_s[None, :], jnp.zeros((77, 128), f32)], axis=0)
    wrb_aug = jnp.concatenate(
        [Wr_b, br_b[None, :], jnp.zeros((77, 128), f32)], axis=0)

    gp512 = pl.pallas_call(
        _group_pos_kernel,
        grid=(n // nb,),
        in_specs=[
            pl.BlockSpec((1, nb, 3), lambda i: (i, 0, 0)),
            pl.BlockSpec((1, 1, nb), lambda i: (i, 0, 0)),
            pl.BlockSpec((1, 1, nb), lambda i: (i, 0, 0)),
        ],
        out_specs=pl.BlockSpec((512, 4), lambda i: (0, 0)),
        out_shape=jax.ShapeDtypeStruct((512, 4), f32),
        scratch_shapes=[pltpu.VMEM((512, 4), f32)],
        name="group_pos",
    )(pos3d, z3d, lab3d)

    mesh = plsc.VectorSubcoreMesh(core_axis_name="c", subcore_axis_name="s")
    ew2m, bw2m, zsrc = pl.kernel(
        functools.partial(_sc_gather_body, n_edges, n_bip),
        out_type=[
            jax.ShapeDtypeStruct((epad,), f32),
            jax.ShapeDtypeStruct((bpad,), f32),
            jax.ShapeDtypeStruct((epad,), jnp.int32),
        ],
        mesh=mesh,
        scratch_types=[
            pltpu.VMEM((n * 4,), f32),
            pltpu.VMEM((n,), jnp.int32),
            pltpu.VMEM((2048,), f32),
            pltpu.VMEM((ept,), jnp.int32),
            pltpu.VMEM((ept,), jnp.int32),
            pltpu.VMEM((ept,), jnp.int32),
            pltpu.VMEM((ept,), f32),
            pltpu.VMEM((bpt,), jnp.int32),
            pltpu.VMEM((bpt,), jnp.int32),
            pltpu.VMEM((bpt,), f32),
        ],
        compiler_params=pltpu.CompilerParams(needs_layout_passes=False),
        name="sc_gather",
    )(srcp, dstp, nidp, gidp, z, pos4.reshape(-1), gp512.reshape(-1))

    msg, eaug = pl.pallas_call(
        _edge_kernel,
        grid=(epad // eblk,),
        in_specs=[
            pl.BlockSpec((1, 1, eblk), lambda i: (i, 0, 0)),
            pl.BlockSpec((1, 1, eblk), lambda i: (i, 0, 0)),
            pl.BlockSpec((128, 128), lambda i: (0, 0)),
            pl.BlockSpec((1, 64), lambda i: (0, 0)),
            pl.BlockSpec((1, 64), lambda i: (0, 0)),
            pl.BlockSpec((64, 128), lambda i: (0, 0)),
            pl.BlockSpec((1, 128), lambda i: (0, 0)),
        ],
        out_specs=[
            pl.BlockSpec((eblk, 128), lambda i: (i, 0)),
            pl.BlockSpec((eblk, 128), lambda i: (i, 0)),
        ],
        out_shape=[
            jax.ShapeDtypeStruct((epad, 128), f32),
            jax.ShapeDtypeStruct((epad, 128), f32),
        ],
        name="edge_rows",
    )(ew2m.reshape(epad // eblk, 1, eblk),
      zsrc.reshape(epad // eblk, 1, eblk), ne_embp, means_sp, betas_sp,
      wdp, ne_bd.reshape(1, 128))

    baug = pl.pallas_call(
        _bip_kernel,
        grid=(bpad // eblk,),
        in_specs=[
            pl.BlockSpec((1, 1, eblk), lambda i: (i, 0, 0)),
            pl.BlockSpec((1, 64), lambda i: (0, 0)),
            pl.BlockSpec((1, 64), lambda i: (0, 0)),
        ],
        out_specs=pl.BlockSpec((eblk, 128), lambda i: (i, 0)),
        out_shape=jax.ShapeDtypeStruct((bpad, 128), f32),
        name="bip_rows",
    )(bw2m.reshape(bpad // eblk, 1, eblk), means_bp, betas_bp)

    def scatter_call(rows, idx, idx_len, tag):
        return pl.kernel(
            functools.partial(_sc_scatter_body, npad),
            out_type=jax.ShapeDtypeStruct((NC, npad, 128), f32),
            mesh=mesh,
            scratch_types=[
                pltpu.VMEM_SHARED((npad, 128), f32),
                pltpu.VMEM((2, 128, 128), f32),
                pltpu.VMEM((idx_len,), jnp.int32),
                pltpu.SemaphoreType.DMA,
                pltpu.SemaphoreType.DMA,
            ],
            name="sc_scatter_" + tag,
        )(rows, idx)

    parts_m = scatter_call(msg, dstp, ept, "m")
    parts_s = scatter_call(eaug, dstp, ept, "s")
    parts_l = scatter_call(baug, nidp, bpt, "l")

    nbd = 2048
    z3d_d = jnp.pad(z, (0, npad - n)).reshape(npad // nbd, 1, nbd)
    out = pl.pallas_call(
        _final_kernel,
        grid=(npad // nbd,),
        in_specs=[
            pl.BlockSpec((NC, nbd, 128), lambda i: (0, i, 0)),
            pl.BlockSpec((NC, nbd, 128), lambda i: (0, i, 0)),
            pl.BlockSpec((NC, nbd, 128), lambda i: (0, i, 0)),
            pl.BlockSpec((1, 1, nbd), lambda i: (i, 0, 0)),
            pl.BlockSpec((128, 128), lambda i: (0, 0)),
            pl.BlockSpec((128, 128), lambda i: (0, 0)),
            pl.BlockSpec((128, 128), lambda i: (0, 0)),
            pl.BlockSpec((1, 128), lambda i: (0, 0)),
            pl.BlockSpec((128, 256), lambda i: (0, 0)),
            pl.BlockSpec((1, 256), lambda i: (0, 0)),
            pl.BlockSpec((128, 128), lambda i: (0, 0)),
            pl.BlockSpec((128, 128), lambda i: (0, 0)),
        ],
        out_specs=pl.BlockSpec((nbd, 256), lambda i: (i, 0)),
        out_shape=jax.ShapeDtypeStruct((npad, 256), f32),
        name="node_final",
    )(parts_m, parts_s, parts_l, z3d_d, embp, ne_Wc[:128], ne_Wc[128:],
      ne_bc.reshape(1, 128), Wm, bm.reshape(1, 256), wrs_aug, wrb_aug)

    return out[:n]


# speedup vs baseline: 1.0636x; 1.0146x over previous
import functools

import jax
import jax.numpy as jnp
from jax import lax
from jax.experimental import pallas as pl
from jax.experimental.pallas import tpu as pltpu
from jax.experimental.pallas import tpu_sc as plsc

CU = 10.0
ALPHA = 5.0 / CU
NC = 2
NS = 16
NW = NC * NS


def _round_up(x, m):
    return (x + m - 1) // m * m


def _group_pos_kernel(pos_ref, z_ref, lab_ref, gp_ref, acc_ref):
    i = pl.program_id(0)

    @pl.when(i == 0)
    def _():
        acc_ref[...] = jnp.zeros_like(acc_ref)

    zf = z_ref[0, 0, :].astype(jnp.float32)
    lab = lab_ref[0, 0, :]
    posb = pos_ref[0]
    w4 = jnp.concatenate([posb * zf[:, None], zf[:, None]], axis=1)
    oh_t = (lax.broadcasted_iota(jnp.int32, (512, lab.shape[0]), 0)
            == lab[None, :]).astype(jnp.float32)
    acc_ref[...] += jnp.dot(oh_t, w4, preferred_element_type=jnp.float32)

    @pl.when(i == pl.num_programs(0) - 1)
    def _():
        acc = acc_ref[...]
        den = acc[:, 3:4]
        gp = acc[:, :3] / jnp.where(den > 0, den, 1.0)
        gp_ref[...] = jnp.concatenate(
            [gp, jnp.zeros((512, 1), jnp.float32)], axis=1)


def _sc_gather_body(n_edges, n_bip, src_hbm, dst_hbm, nid_hbm, gid_hbm,
                    z_hbm, pos_hbm, gp_hbm,
                    ew2_out, bw2_out, zsrc_out,
                    pos_v, z_v, gp_v, src_v, dst_v, zsrc_v, ew2_v,
                    nid_v, gid_v, bw2_v):
    ept = src_v.shape[0]
    bpt = nid_v.shape[0]
    wid = lax.axis_index("c") * NS + lax.axis_index("s")
    base = wid * ept
    bbase = wid * bpt

    pltpu.sync_copy(pos_hbm, pos_v)
    pltpu.sync_copy(z_hbm, z_v)
    pltpu.sync_copy(gp_hbm, gp_v)
    pltpu.sync_copy(src_hbm.at[pl.ds(base, ept)], src_v)
    pltpu.sync_copy(dst_hbm.at[pl.ds(base, ept)], dst_v)
    pltpu.sync_copy(nid_hbm.at[pl.ds(bbase, bpt)], nid_v)
    pltpu.sync_copy(gid_hbm.at[pl.ds(bbase, bpt)], gid_v)

    lane = lax.iota(jnp.int32, 16)

    def edge_body(k, _):
        s4 = src_v[pl.ds(k * 16, 16)] * 4
        d4 = dst_v[pl.ds(k * 16, 16)] * 4
        dx = plsc.load_gather(pos_v, [s4]) - plsc.load_gather(pos_v, [d4])
        dy = (plsc.load_gather(pos_v, [s4 + 1])
              - plsc.load_gather(pos_v, [d4 + 1]))
        dz = (plsc.load_gather(pos_v, [s4 + 2])
              - plsc.load_gather(pos_v, [d4 + 2]))
        ew2 = dx * dx + dy * dy + dz * dz
        ge = base + k * 16 + lane
        ok = (ge < n_edges) & (s4 != d4)
        ew2_v[pl.ds(k * 16, 16)] = jnp.where(ok, ew2, -1.0)
        zsrc_v[pl.ds(k * 16, 16)] = plsc.load_gather(
            z_v, [src_v[pl.ds(k * 16, 16)]])
        return 0

    lax.fori_loop(0, ept // 16, edge_body, 0)

    def bip_body(k, _):
        n4 = nid_v[pl.ds(k * 16, 16)] * 4
        g4 = gid_v[pl.ds(k * 16, 16)] * 4
        dx = plsc.load_gather(pos_v, [n4]) - plsc.load_gather(gp_v, [g4])
        dy = (plsc.load_gather(pos_v, [n4 + 1])
              - plsc.load_gather(gp_v, [g4 + 1]))
        dz = (plsc.load_gather(pos_v, [n4 + 2])
              - plsc.load_gather(gp_v, [g4 + 2]))
        bw2 = dx * dx + dy * dy + dz * dz
        ge = bbase + k * 16 + lane
        bw2_v[pl.ds(k * 16, 16)] = jnp.where(ge < n_bip, bw2, 1e9)
        return 0

    lax.fori_loop(0, bpt // 16, bip_body, 0)

    pltpu.sync_copy(ew2_v, ew2_out.at[pl.ds(base, ept)])
    pltpu.sync_copy(bw2_v, bw2_out.at[pl.ds(bbase, bpt)])
    pltpu.sync_copy(zsrc_v, zsrc_out.at[pl.ds(base, ept)])


def _edge_kernel(ew2_ref, zsrc_ref, emb_ref, means_ref, betas_ref, wd_ref,
                 bd_ref, msg_ref, eaug_ref):
    ew2m = ew2_ref[0, 0, :]
    zsrc = zsrc_ref[0, 0, :]
    oh = (lax.broadcasted_iota(jnp.int32, (zsrc.shape[0], 128), 1)
          == zsrc[:, None]).astype(jnp.float32)
    xsrc = jnp.dot(oh, emb_ref[...], preferred_element_type=jnp.float32)
    kf = (ew2m >= 0).astype(jnp.float32)
    ew = jnp.sqrt(jnp.maximum(ew2m, 0.0))
    cc = jnp.where(ew < CU, 0.5 * (jnp.cos(jnp.pi * ew / CU) + 1.0), 0.0)
    u = jnp.exp(-ALPHA * ew)
    t = u[:, None] - means_ref[0, :][None, :]
    ea = cc[:, None] * jnp.exp(-betas_ref[0, :][None, :] * t * t)
    cvec = cc * kf
    wmsg = (jnp.dot(ea, wd_ref[...], preferred_element_type=jnp.float32)
            + bd_ref[0, :][None, :]) * cvec[:, None]
    msg_ref[...] = xsrc * wmsg
    col = lax.broadcasted_iota(jnp.int32, ea.shape, 1)
    part = jnp.where(col < 50, ea * kf[:, None],
                     jnp.where(col == 50, kf[:, None], 0.0))
    eaug_ref[...] = jnp.concatenate(
        [part, jnp.zeros_like(part)], axis=1)


def _bip_kernel(bw2_ref, means_ref, betas_ref, out_ref):
    bw2 = bw2_ref[0, 0, :]
    bw = jnp.sqrt(bw2)
    bf = (bw <= CU).astype(jnp.float32)
    cc = jnp.where(bw < CU, 0.5 * (jnp.cos(jnp.pi * bw / CU) + 1.0), 0.0)
    u = jnp.exp(-ALPHA * bw)
    t = u[:, None] - means_ref[0, :][None, :]
    ea = cc[:, None] * jnp.exp(-betas_ref[0, :][None, :] * t * t)
    col = lax.broadcasted_iota(jnp.int32, ea.shape, 1)
    part = jnp.where(col < 50, ea * bf[:, None],
                     jnp.where(col == 50, bf[:, None], 0.0))
    out_ref[...] = jnp.concatenate([part, jnp.zeros_like(part)], axis=1)


def _sc_scatter_body(n_rows, rows_hbm, idx_hbm, out_hbm, acc, buf, idx_v,
                     sem_r, sem_s):
    ch = buf.shape[1]
    ept = idx_v.shape[0]
    c = lax.axis_index("c")
    s = lax.axis_index("s")
    wid = c * NS + s
    rows_per_tile = n_rows // NS
    lane = lax.iota(jnp.int32, 16)

    def bfill(i, _):
        r = i // 8
        cc = (i % 8) * 16
        buf[0, r, pl.ds(cc, 16)] = jnp.zeros((16,), jnp.float32)
        return 0

    lax.fori_loop(0, 128 * 8, bfill, 0)

    def ifill(i, _):
        idx_v[pl.ds(i * 16, 16)] = s * rows_per_tile + i * 16 + lane
        return 0

    lax.fori_loop(0, rows_per_tile // 16, ifill, 0)

    def zcp(q, _):
        pltpu.sync_copy(buf.at[0],
                        acc.at[idx_v.at[pl.ds(q * 128, 128)]])
        return 0

    lax.fori_loop(0, rows_per_tile // 128, zcp, 0)
    plsc.subcore_barrier()
    pltpu.sync_copy(idx_hbm.at[pl.ds(wid * ept, ept)], idx_v)

    nk = ept // ch

    def r_desc(k, b):
        return pltpu.make_async_copy(
            rows_hbm.at[pl.ds(wid * ept + k * ch, ch)], buf.at[b], sem_r)

    def s_desc(k, b):
        return pltpu.make_async_copy(
            buf.at[b], acc.at[idx_v.at[pl.ds(k * ch, ch)]], sem_s)

    def s_start(k, b):
        pltpu.async_copy(buf.at[b], acc.at[idx_v.at[pl.ds(k * ch, ch)]],
                         sem_s, add=True)

    r_desc(0, 0).start()

    def pair_body(j, _):
        k0 = 2 * j

        @pl.when(j >= 1)
        def _():
            s_desc(k0 - 1, 1).wait()

        r_desc(k0 + 1, 1).start()
        r_desc(k0, 0).wait()
        s_start(k0, 0)

        @pl.when(k0 + 2 < nk)
        def _():
            s_desc(k0, 0).wait()
            r_desc(k0 + 2, 0).start()

        r_desc(k0 + 1, 1).wait()
        s_start(k0 + 1, 1)
        return 0

    lax.fori_loop(0, nk // 2, pair_body, 0)
    s_desc(nk - 2, 0).wait()
    s_desc(nk - 1, 1).wait()
    plsc.subcore_barrier()
    pltpu.sync_copy(acc.at[pl.ds(s * rows_per_tile, rows_per_tile)],
                    out_hbm.at[c, pl.ds(s * rows_per_tile, rows_per_tile)])


def _final_kernel(pma_ref, pmb_ref, psa_ref, psb_ref, pl_ref, z_ref,
                  emb_ref, wct_ref, wcb_ref, bc_ref, wm_ref, bm_ref,
                  wrs_ref, wrb_ref, out_ref):
    agg = (pma_ref[0] + pma_ref[1]) + (pmb_ref[0] + pmb_ref[1])
    sacc = (psa_ref[0] + psa_ref[1]) + (psb_ref[0] + psb_ref[1])
    acc2 = pl_ref[0] + pl_ref[1]
    zb = z_ref[0, 0, :]
    oh = (lax.broadcasted_iota(jnp.int32, (zb.shape[0], 128), 1)
          == zb[:, None]).astype(jnp.float32)
    nx = jnp.dot(oh, emb_ref[...], preferred_element_type=jnp.float32)
    h = (jnp.dot(nx, wct_ref[...], preferred_element_type=jnp.float32)
         + jnp.dot(agg, wcb_ref[...], preferred_element_type=jnp.float32)
         + bc_ref[0, :][None, :])
    node_cat = (jnp.dot(h, wm_ref[...], preferred_element_type=jnp.float32)
                + bm_ref[0, :][None, :])
    out_s = jnp.dot(sacc, wrs_ref[...], preferred_element_type=jnp.float32)
    out_l = jnp.dot(acc2, wrb_ref[...], preferred_element_type=jnp.float32)
    out_ref[...] = node_cat + jnp.concatenate([out_s, out_l], axis=1)


def kernel(z, pos, labels, edge_index, interaction_graph, emb, ne_emb,
           ne_Wd, ne_bd, ne_Wc, ne_bc, means_s, betas_s, Wr_s, br_s,
           means_b, betas_b, Wr_b, br_b, Wm, bm):
    n = z.shape[0]
    n_edges = edge_index.shape[1]
    n_bip = interaction_graph.shape[1]
    f32 = jnp.float32

    npad = _round_up(n, 2048)
    epad = _round_up(n_edges, 8192)
    bpad = _round_up(n_bip, 8192)
    ept = epad // NW
    bpt = bpad // NW
    nb = 2000
    eblk = 2048

    srcp = jnp.pad(edge_index[0], (0, epad - n_edges))
    dstp = jnp.pad(edge_index[1], (0, epad - n_edges))
    nidp = jnp.pad(interaction_graph[0], (0, bpad - n_bip))
    gidp = jnp.pad(interaction_graph[1], (0, bpad - n_bip))
    pos4 = jnp.pad(pos, ((0, 0), (0, 1)))
    z3d = z.reshape(n // nb, 1, nb)
    lab3d = labels.reshape(n // nb, 1, nb)
    pos3d = pos.reshape(n // nb, nb, 3)
    means_sp = jnp.pad(means_s, (0, 14)).reshape(1, 64)
    betas_sp = jnp.pad(betas_s, (0, 14)).reshape(1, 64)
    means_bp = jnp.pad(means_b, (0, 14)).reshape(1, 64)
    betas_bp = jnp.pad(betas_b, (0, 14)).reshape(1, 64)
    wdp = jnp.pad(ne_Wd, ((0, 14), (0, 0)))
    embp = jnp.pad(emb, ((0, 128 - emb.shape[0]), (0, 0)))
    ne_embp = jnp.pad(ne_emb, ((0, 128 - ne_emb.shape[0]), (0, 0)))
    wrs_aug = jnp.concatenate(
        [Wr_s, br_s[None, :], jnp.zeros((77, 128), f32)], axis=0)
    wrb_aug = jnp.concatenate(
        [Wr_b, br_b[None, :], jnp.zeros((77, 128), f32)], axis=0)

    gp512 = pl.pallas_call(
        _group_pos_kernel,
        grid=(n // nb,),
        in_specs=[
            pl.BlockSpec((1, nb, 3), lambda i: (i, 0, 0)),
            pl.BlockSpec((1, 1, nb), lambda i: (i, 0, 0)),
            pl.BlockSpec((1, 1, nb), lambda i: (i, 0, 0)),
        ],
        out_specs=pl.BlockSpec((512, 4), lambda i: (0, 0)),
        out_shape=jax.ShapeDtypeStruct((512, 4), f32),
        scratch_shapes=[pltpu.VMEM((512, 4), f32)],
        name="group_pos",
    )(pos3d, z3d, lab3d)

    mesh = plsc.VectorSubcoreMesh(core_axis_name="c", subcore_axis_name="s")
    ew2m, bw2m, zsrc = pl.kernel(
        functools.partial(_sc_gather_body, n_edges, n_bip),
        out_type=[
            jax.ShapeDtypeStruct((epad,), f32),
            jax.ShapeDtypeStruct((bpad,), f32),
            jax.ShapeDtypeStruct((epad,), jnp.int32),
        ],
        mesh=mesh,
        scratch_types=[
            pltpu.VMEM((n * 4,), f32),
            pltpu.VMEM((n,), jnp.int32),
            pltpu.VMEM((2048,), f32),
            pltpu.VMEM((ept,), jnp.int32),
            pltpu.VMEM((ept,), jnp.int32),
            pltpu.VMEM((ept,), jnp.int32),
            pltpu.VMEM((ept,), f32),
            pltpu.VMEM((bpt,), jnp.int32),
            pltpu.VMEM((bpt,), jnp.int32),
            pltpu.VMEM((bpt,), f32),
        ],
        compiler_params=pltpu.CompilerParams(needs_layout_passes=False),
        name="sc_gather",
    )(srcp, dstp, nidp, gidp, z, pos4.reshape(-1), gp512.reshape(-1))

    eh = epad // 2

    def edge_call(ew2_half, zsrc_half, tag):
        return pl.pallas_call(
            _edge_kernel,
            grid=(eh // eblk,),
            in_specs=[
                pl.BlockSpec((1, 1, eblk), lambda i: (i, 0, 0)),
                pl.BlockSpec((1, 1, eblk), lambda i: (i, 0, 0)),
                pl.BlockSpec((128, 128), lambda i: (0, 0)),
                pl.BlockSpec((1, 64), lambda i: (0, 0)),
                pl.BlockSpec((1, 64), lambda i: (0, 0)),
                pl.BlockSpec((64, 128), lambda i: (0, 0)),
                pl.BlockSpec((1, 128), lambda i: (0, 0)),
            ],
            out_specs=[
                pl.BlockSpec((eblk, 128), lambda i: (i, 0)),
                pl.BlockSpec((eblk, 128), lambda i: (i, 0)),
            ],
            out_shape=[
                jax.ShapeDtypeStruct((eh, 128), f32),
                jax.ShapeDtypeStruct((eh, 128), f32),
            ],
            name="edge_rows_" + tag,
        )(ew2_half.reshape(eh // eblk, 1, eblk),
          zsrc_half.reshape(eh // eblk, 1, eblk), ne_embp, means_sp,
          betas_sp, wdp, ne_bd.reshape(1, 128))

    msg_a, eaug_a = edge_call(ew2m[:eh], zsrc[:eh], "a")
    msg_b, eaug_b = edge_call(ew2m[eh:], zsrc[eh:], "b")

    baug = pl.pallas_call(
        _bip_kernel,
        grid=(bpad // eblk,),
        in_specs=[
            pl.BlockSpec((1, 1, eblk), lambda i: (i, 0, 0)),
            pl.BlockSpec((1, 64), lambda i: (0, 0)),
            pl.BlockSpec((1, 64), lambda i: (0, 0)),
        ],
        out_specs=pl.BlockSpec((eblk, 128), lambda i: (i, 0)),
        out_shape=jax.ShapeDtypeStruct((bpad, 128), f32),
        name="bip_rows",
    )(bw2m.reshape(bpad // eblk, 1, eblk), means_bp, betas_bp)

    def scatter_call(rows, idx, idx_len, tag):
        return pl.kernel(
            functools.partial(_sc_scatter_body, npad),
            out_type=jax.ShapeDtypeStruct((NC, npad, 128), f32),
            mesh=mesh,
            scratch_types=[
                pltpu.VMEM_SHARED((npad, 128), f32),
                pltpu.VMEM((2, 128, 128), f32),
                pltpu.VMEM((idx_len,), jnp.int32),
                pltpu.SemaphoreType.DMA,
                pltpu.SemaphoreType.DMA,
            ],
            name="sc_scatter_" + tag,
        )(rows, idx)

    parts_ma = scatter_call(msg_a, dstp[:eh], eh // NW, "ma")
    parts_sa = scatter_call(eaug_a, dstp[:eh], eh // NW, "sa")
    parts_mb = scatter_call(msg_b, dstp[eh:], eh // NW, "mb")
    parts_sb = scatter_call(eaug_b, dstp[eh:], eh // NW, "sb")
    parts_l = scatter_call(baug, nidp, bpt, "l")

    nbd = 2048
    z3d_d = jnp.pad(z, (0, npad - n)).reshape(npad // nbd, 1, nbd)
    out = pl.pallas_call(
        _final_kernel,
        grid=(npad // nbd,),
        in_specs=[
            pl.BlockSpec((NC, nbd, 128), lambda i: (0, i, 0)),
            pl.BlockSpec((NC, nbd, 128), lambda i: (0, i, 0)),
            pl.BlockSpec((NC, nbd, 128), lambda i: (0, i, 0)),
            pl.BlockSpec((NC, nbd, 128), lambda i: (0, i, 0)),
            pl.BlockSpec((NC, nbd, 128), lambda i: (0, i, 0)),
            pl.BlockSpec((1, 1, nbd), lambda i: (i, 0, 0)),
            pl.BlockSpec((128, 128), lambda i: (0, 0)),
            pl.BlockSpec((128, 128), lambda i: (0, 0)),
            pl.BlockSpec((128, 128), lambda i: (0, 0)),
            pl.BlockSpec((1, 128), lambda i: (0, 0)),
            pl.BlockSpec((128, 256), lambda i: (0, 0)),
            pl.BlockSpec((1, 256), lambda i: (0, 0)),
            pl.BlockSpec((128, 128), lambda i: (0, 0)),
            pl.BlockSpec((128, 128), lambda i: (0, 0)),
        ],
        out_specs=pl.BlockSpec((nbd, 256), lambda i: (i, 0)),
        out_shape=jax.ShapeDtypeStruct((npad, 256), f32),
        name="node_final",
    )(parts_ma, parts_mb, parts_sa, parts_sb, parts_l, z3d_d, embp,
      ne_Wc[:128], ne_Wc[128:], ne_bc.reshape(1, 128), Wm,
      bm.reshape(1, 256), wrs_aug, wrb_aug)

    return out[:n]


# speedup vs baseline: 1.1120x; 1.0455x over previous
import functools

import jax
import jax.numpy as jnp
from jax import lax
from jax.experimental import pallas as pl
from jax.experimental.pallas import tpu as pltpu
from jax.experimental.pallas import tpu_sc as plsc

CU = 10.0
ALPHA = 5.0 / CU
NC = 2
NS = 16
NW = NC * NS


def _round_up(x, m):
    return (x + m - 1) // m * m


def _group_pos_kernel(pos_ref, z_ref, lab_ref, gp_ref, acc_ref):
    i = pl.program_id(0)

    @pl.when(i == 0)
    def _():
        acc_ref[...] = jnp.zeros_like(acc_ref)

    zf = z_ref[0, 0, :].astype(jnp.float32)
    lab = lab_ref[0, 0, :]
    posb = pos_ref[0]
    w4 = jnp.concatenate([posb * zf[:, None], zf[:, None]], axis=1)
    oh_t = (lax.broadcasted_iota(jnp.int32, (512, lab.shape[0]), 0)
            == lab[None, :]).astype(jnp.float32)
    acc_ref[...] += jnp.dot(oh_t, w4, preferred_element_type=jnp.float32)

    @pl.when(i == pl.num_programs(0) - 1)
    def _():
        acc = acc_ref[...]
        den = acc[:, 3:4]
        gp = acc[:, :3] / jnp.where(den > 0, den, 1.0)
        gp_ref[...] = jnp.concatenate(
            [gp, jnp.zeros((512, 1), jnp.float32)], axis=1)


def _sc_gather_body(n_edges, n_bip, src_hbm, dst_hbm, nid_hbm, gid_hbm,
                    z_hbm, pos_hbm, gp_hbm,
                    ew2_out, bw2_out, zsrc_out,
                    pos_v, z_v, gp_v, src_v, dst_v, zsrc_v, ew2_v,
                    nid_v, gid_v, bw2_v):
    ept = src_v.shape[0]
    bpt = nid_v.shape[0]
    wid = lax.axis_index("c") * NS + lax.axis_index("s")
    base = wid * ept
    bbase = wid * bpt

    pltpu.sync_copy(pos_hbm, pos_v)
    pltpu.sync_copy(z_hbm, z_v)
    pltpu.sync_copy(gp_hbm, gp_v)
    pltpu.sync_copy(src_hbm.at[pl.ds(base, ept)], src_v)
    pltpu.sync_copy(dst_hbm.at[pl.ds(base, ept)], dst_v)
    pltpu.sync_copy(nid_hbm.at[pl.ds(bbase, bpt)], nid_v)
    pltpu.sync_copy(gid_hbm.at[pl.ds(bbase, bpt)], gid_v)

    lane = lax.iota(jnp.int32, 16)

    def edge_body(k, _):
        s4 = src_v[pl.ds(k * 16, 16)] * 4
        d4 = dst_v[pl.ds(k * 16, 16)] * 4
        dx = plsc.load_gather(pos_v, [s4]) - plsc.load_gather(pos_v, [d4])
        dy = (plsc.load_gather(pos_v, [s4 + 1])
              - plsc.load_gather(pos_v, [d4 + 1]))
        dz = (plsc.load_gather(pos_v, [s4 + 2])
              - plsc.load_gather(pos_v, [d4 + 2]))
        ew2 = dx * dx + dy * dy + dz * dz
        ge = base + k * 16 + lane
        ok = (ge < n_edges) & (s4 != d4)
        ew2_v[pl.ds(k * 16, 16)] = jnp.where(ok, ew2, -1.0)
        zsrc_v[pl.ds(k * 16, 16)] = plsc.load_gather(
            z_v, [src_v[pl.ds(k * 16, 16)]])
        return 0

    lax.fori_loop(0, ept // 16, edge_body, 0)

    def bip_body(k, _):
        n4 = nid_v[pl.ds(k * 16, 16)] * 4
        g4 = gid_v[pl.ds(k * 16, 16)] * 4
        dx = plsc.load_gather(pos_v, [n4]) - plsc.load_gather(gp_v, [g4])
        dy = (plsc.load_gather(pos_v, [n4 + 1])
              - plsc.load_gather(gp_v, [g4 + 1]))
        dz = (plsc.load_gather(pos_v, [n4 + 2])
              - plsc.load_gather(gp_v, [g4 + 2]))
        bw2 = dx * dx + dy * dy + dz * dz
        ge = bbase + k * 16 + lane
        bw2_v[pl.ds(k * 16, 16)] = jnp.where(ge < n_bip, bw2, 1e9)
        return 0

    lax.fori_loop(0, bpt // 16, bip_body, 0)

    pltpu.sync_copy(ew2_v, ew2_out.at[pl.ds(base, ept)])
    pltpu.sync_copy(bw2_v, bw2_out.at[pl.ds(bbase, bpt)])
    pltpu.sync_copy(zsrc_v, zsrc_out.at[pl.ds(base, ept)])


def _edge_kernel(ew2_ref, zsrc_ref, emb_ref, means_ref, betas_ref, wd_ref,
                 bd_ref, msg_ref, eaug_ref):
    ew2m = ew2_ref[0, 0, :]
    zsrc = zsrc_ref[0, 0, :]
    oh = (lax.broadcasted_iota(jnp.int32, (zsrc.shape[0], 128), 1)
          == zsrc[:, None]).astype(jnp.float32)
    xsrc = jnp.dot(oh, emb_ref[...], preferred_element_type=jnp.float32)
    kf = (ew2m >= 0).astype(jnp.float32)
    ew = jnp.sqrt(jnp.maximum(ew2m, 0.0))
    cc = jnp.where(ew < CU, 0.5 * (jnp.cos(jnp.pi * ew / CU) + 1.0), 0.0)
    u = jnp.exp(-ALPHA * ew)
    t = u[:, None] - means_ref[0, :][None, :]
    ea = cc[:, None] * jnp.exp(-betas_ref[0, :][None, :] * t * t)
    cvec = cc * kf
    wmsg = (jnp.dot(ea, wd_ref[...], preferred_element_type=jnp.float32)
            + bd_ref[0, :][None, :]) * cvec[:, None]
    msg_ref[...] = xsrc * wmsg
    col = lax.broadcasted_iota(jnp.int32, ea.shape, 1)
    eaug_ref[...] = jnp.where(col < 50, ea * kf[:, None],
                              jnp.where(col == 50, kf[:, None], 0.0))


def _bip_kernel(bw2_ref, means_ref, betas_ref, out_ref):
    bw2 = bw2_ref[0, 0, :]
    bw = jnp.sqrt(bw2)
    bf = (bw <= CU).astype(jnp.float32)
    cc = jnp.where(bw < CU, 0.5 * (jnp.cos(jnp.pi * bw / CU) + 1.0), 0.0)
    u = jnp.exp(-ALPHA * bw)
    t = u[:, None] - means_ref[0, :][None, :]
    ea = cc[:, None] * jnp.exp(-betas_ref[0, :][None, :] * t * t)
    col = lax.broadcasted_iota(jnp.int32, ea.shape, 1)
    out_ref[...] = jnp.where(col < 50, ea * bf[:, None],
                             jnp.where(col == 50, bf[:, None], 0.0))


def _sc_scatter_body(n_rows, rows_hbm, idx_hbm, out_hbm, acc, buf, idx_v,
                     sem_r, sem_s):
    ch = buf.shape[1]
    ept = idx_v.shape[0]
    c = lax.axis_index("c")
    s = lax.axis_index("s")
    wid = c * NS + s
    rows_per_tile = n_rows // NS
    lane = lax.iota(jnp.int32, 16)

    width = buf.shape[2]

    def bfill(i, _):
        r = i // (width // 16)
        cc = (i % (width // 16)) * 16
        buf[0, r, pl.ds(cc, 16)] = jnp.zeros((16,), jnp.float32)
        return 0

    lax.fori_loop(0, 128 * (width // 16), bfill, 0)

    def ifill(i, _):
        idx_v[pl.ds(i * 16, 16)] = s * rows_per_tile + i * 16 + lane
        return 0

    lax.fori_loop(0, rows_per_tile // 16, ifill, 0)

    def zcp(q, _):
        pltpu.sync_copy(buf.at[0],
                        acc.at[idx_v.at[pl.ds(q * 128, 128)]])
        return 0

    lax.fori_loop(0, rows_per_tile // 128, zcp, 0)
    plsc.subcore_barrier()
    pltpu.sync_copy(idx_hbm.at[pl.ds(wid * ept, ept)], idx_v)

    nk = ept // ch

    def r_desc(k, b):
        return pltpu.make_async_copy(
            rows_hbm.at[pl.ds(wid * ept + k * ch, ch)], buf.at[b], sem_r)

    def s_desc(k, b):
        return pltpu.make_async_copy(
            buf.at[b], acc.at[idx_v.at[pl.ds(k * ch, ch)]], sem_s)

    def s_start(k, b):
        pltpu.async_copy(buf.at[b], acc.at[idx_v.at[pl.ds(k * ch, ch)]],
                         sem_s, add=True)

    r_desc(0, 0).start()

    def pair_body(j, _):
        k0 = 2 * j

        @pl.when(j >= 1)
        def _():
            s_desc(k0 - 1, 1).wait()

        r_desc(k0 + 1, 1).start()
        r_desc(k0, 0).wait()
        s_start(k0, 0)

        @pl.when(k0 + 2 < nk)
        def _():
            s_desc(k0, 0).wait()
            r_desc(k0 + 2, 0).start()

        r_desc(k0 + 1, 1).wait()
        s_start(k0 + 1, 1)
        return 0

    lax.fori_loop(0, nk // 2, pair_body, 0)
    s_desc(nk - 2, 0).wait()
    s_desc(nk - 1, 1).wait()
    plsc.subcore_barrier()
    pltpu.sync_copy(acc.at[pl.ds(s * rows_per_tile, rows_per_tile)],
                    out_hbm.at[c, pl.ds(s * rows_per_tile, rows_per_tile)])


def _final_kernel(pma_ref, pmb_ref, psa_ref, psb_ref, pl_ref, z_ref,
                  emb_ref, wct_ref, wcb_ref, bc_ref, wm_ref, bm_ref,
                  wrs_ref, wrb_ref, out_ref):
    agg = (pma_ref[0] + pma_ref[1]) + (pmb_ref[0] + pmb_ref[1])
    sacc = (psa_ref[0] + psa_ref[1]) + (psb_ref[0] + psb_ref[1])
    acc2 = pl_ref[0] + pl_ref[1]
    zb = z_ref[0, 0, :]
    oh = (lax.broadcasted_iota(jnp.int32, (zb.shape[0], 128), 1)
          == zb[:, None]).astype(jnp.float32)
    nx = jnp.dot(oh, emb_ref[...], preferred_element_type=jnp.float32)
    h = (jnp.dot(nx, wct_ref[...], preferred_element_type=jnp.float32)
         + jnp.dot(agg, wcb_ref[...], preferred_element_type=jnp.float32)
         + bc_ref[0, :][None, :])
    node_cat = (jnp.dot(h, wm_ref[...], preferred_element_type=jnp.float32)
                + bm_ref[0, :][None, :])
    out_s = jnp.dot(sacc, wrs_ref[...], preferred_element_type=jnp.float32)
    out_l = jnp.dot(acc2, wrb_ref[...], preferred_element_type=jnp.float32)
    out_ref[...] = node_cat + jnp.concatenate([out_s, out_l], axis=1)


def kernel(z, pos, labels, edge_index, interaction_graph, emb, ne_emb,
           ne_Wd, ne_bd, ne_Wc, ne_bc, means_s, betas_s, Wr_s, br_s,
           means_b, betas_b, Wr_b, br_b, Wm, bm):
    n = z.shape[0]
    n_edges = edge_index.shape[1]
    n_bip = interaction_graph.shape[1]
    f32 = jnp.float32

    npad = _round_up(n, 2048)
    epad = _round_up(n_edges, 8192)
    bpad = _round_up(n_bip, 8192)
    ept = epad // NW
    bpt = bpad // NW
    nb = 2000
    eblk = 2048

    srcp = jnp.pad(edge_index[0], (0, epad - n_edges))
    dstp = jnp.pad(edge_index[1], (0, epad - n_edges))
    nidp = jnp.pad(interaction_graph[0], (0, bpad - n_bip))
    gidp = jnp.pad(interaction_graph[1], (0, bpad - n_bip))
    pos4 = jnp.pad(pos, ((0, 0), (0, 1)))
    z3d = z.reshape(n // nb, 1, nb)
    lab3d = labels.reshape(n // nb, 1, nb)
    pos3d = pos.reshape(n // nb, nb, 3)
    means_sp = jnp.pad(means_s, (0, 14)).reshape(1, 64)
    betas_sp = jnp.pad(betas_s, (0, 14)).reshape(1, 64)
    means_bp = jnp.pad(means_b, (0, 14)).reshape(1, 64)
    betas_bp = jnp.pad(betas_b, (0, 14)).reshape(1, 64)
    wdp = jnp.pad(ne_Wd, ((0, 14), (0, 0)))
    embp = jnp.pad(emb, ((0, 128 - emb.shape[0]), (0, 0)))
    ne_embp = jnp.pad(ne_emb, ((0, 128 - ne_emb.shape[0]), (0, 0)))
    wrs_aug = jnp.concatenate(
        [Wr_s, br_s[None, :], jnp.zeros((13, 128), f32)], axis=0)
    wrb_aug = jnp.concatenate(
        [Wr_b, br_b[None, :], jnp.zeros((13, 128), f32)], axis=0)

    gp512 = pl.pallas_call(
        _group_pos_kernel,
        grid=(n // nb,),
        in_specs=[
            pl.BlockSpec((1, nb, 3), lambda i: (i, 0, 0)),
            pl.BlockSpec((1, 1, nb), lambda i: (i, 0, 0)),
            pl.BlockSpec((1, 1, nb), lambda i: (i, 0, 0)),
        ],
        out_specs=pl.BlockSpec((512, 4), lambda i: (0, 0)),
        out_shape=jax.ShapeDtypeStruct((512, 4), f32),
        scratch_shapes=[pltpu.VMEM((512, 4), f32)],
        name="group_pos",
    )(pos3d, z3d, lab3d)

    mesh = plsc.VectorSubcoreMesh(core_axis_name="c", subcore_axis_name="s")
    ew2m, bw2m, zsrc = pl.kernel(
        functools.partial(_sc_gather_body, n_edges, n_bip),
        out_type=[
            jax.ShapeDtypeStruct((epad,), f32),
            jax.ShapeDtypeStruct((bpad,), f32),
            jax.ShapeDtypeStruct((epad,), jnp.int32),
        ],
        mesh=mesh,
        scratch_types=[
            pltpu.VMEM((n * 4,), f32),
            pltpu.VMEM((n,), jnp.int32),
            pltpu.VMEM((2048,), f32),
            pltpu.VMEM((ept,), jnp.int32),
            pltpu.VMEM((ept,), jnp.int32),
            pltpu.VMEM((ept,), jnp.int32),
            pltpu.VMEM((ept,), f32),
            pltpu.VMEM((bpt,), jnp.int32),
            pltpu.VMEM((bpt,), jnp.int32),
            pltpu.VMEM((bpt,), f32),
        ],
        compiler_params=pltpu.CompilerParams(needs_layout_passes=False),
        name="sc_gather",
    )(srcp, dstp, nidp, gidp, z, pos4.reshape(-1), gp512.reshape(-1))

    eh = epad // 2

    def edge_call(ew2_half, zsrc_half, tag):
        return pl.pallas_call(
            _edge_kernel,
            grid=(eh // eblk,),
            in_specs=[
                pl.BlockSpec((1, 1, eblk), lambda i: (i, 0, 0)),
                pl.BlockSpec((1, 1, eblk), lambda i: (i, 0, 0)),
                pl.BlockSpec((128, 128), lambda i: (0, 0)),
                pl.BlockSpec((1, 64), lambda i: (0, 0)),
                pl.BlockSpec((1, 64), lambda i: (0, 0)),
                pl.BlockSpec((64, 128), lambda i: (0, 0)),
                pl.BlockSpec((1, 128), lambda i: (0, 0)),
            ],
            out_specs=[
                pl.BlockSpec((eblk, 128), lambda i: (i, 0)),
                pl.BlockSpec((eblk, 64), lambda i: (i, 0)),
            ],
            out_shape=[
                jax.ShapeDtypeStruct((eh, 128), f32),
                jax.ShapeDtypeStruct((eh, 64), f32),
            ],
            name="edge_rows_" + tag,
        )(ew2_half.reshape(eh // eblk, 1, eblk),
          zsrc_half.reshape(eh // eblk, 1, eblk), ne_embp, means_sp,
          betas_sp, wdp, ne_bd.reshape(1, 128))

    msg_a, eaug_a = edge_call(ew2m[:eh], zsrc[:eh], "a")
    msg_b, eaug_b = edge_call(ew2m[eh:], zsrc[eh:], "b")

    baug = pl.pallas_call(
        _bip_kernel,
        grid=(bpad // eblk,),
        in_specs=[
            pl.BlockSpec((1, 1, eblk), lambda i: (i, 0, 0)),
            pl.BlockSpec((1, 64), lambda i: (0, 0)),
            pl.BlockSpec((1, 64), lambda i: (0, 0)),
        ],
        out_specs=pl.BlockSpec((eblk, 64), lambda i: (i, 0)),
        out_shape=jax.ShapeDtypeStruct((bpad, 64), f32),
        name="bip_rows",
    )(bw2m.reshape(bpad // eblk, 1, eblk), means_bp, betas_bp)

    def scatter_call(rows, idx, idx_len, width, tag):
        return pl.kernel(
            functools.partial(_sc_scatter_body, npad),
            out_type=jax.ShapeDtypeStruct((NC, npad, width), f32),
            mesh=mesh,
            scratch_types=[
                pltpu.VMEM_SHARED((npad, width), f32),
                pltpu.VMEM((2, 128, width), f32),
                pltpu.VMEM((idx_len,), jnp.int32),
                pltpu.SemaphoreType.DMA,
                pltpu.SemaphoreType.DMA,
            ],
            name="sc_scatter_" + tag,
        )(rows, idx)

    parts_ma = scatter_call(msg_a, dstp[:eh], eh // NW, 128, "ma")
    parts_sa = scatter_call(eaug_a, dstp[:eh], eh // NW, 64, "sa")
    parts_mb = scatter_call(msg_b, dstp[eh:], eh // NW, 128, "mb")
    parts_sb = scatter_call(eaug_b, dstp[eh:], eh // NW, 64, "sb")
    parts_l = scatter_call(baug, nidp, bpt, 64, "l")

    nbd = 2048
    z3d_d = jnp.pad(z, (0, npad - n)).reshape(npad // nbd, 1, nbd)
    out = pl.pallas_call(
        _final_kernel,
        grid=(npad // nbd,),
        in_specs=[
            pl.BlockSpec((NC, nbd, 128), lambda i: (0, i, 0)),
            pl.BlockSpec((NC, nbd, 128), lambda i: (0, i, 0)),
            pl.BlockSpec((NC, nbd, 64), lambda i: (0, i, 0)),
            pl.BlockSpec((NC, nbd, 64), lambda i: (0, i, 0)),
            pl.BlockSpec((NC, nbd, 64), lambda i: (0, i, 0)),
            pl.BlockSpec((1, 1, nbd), lambda i: (i, 0, 0)),
            pl.BlockSpec((128, 128), lambda i: (0, 0)),
            pl.BlockSpec((128, 128), lambda i: (0, 0)),
            pl.BlockSpec((128, 128), lambda i: (0, 0)),
            pl.BlockSpec((1, 128), lambda i: (0, 0)),
            pl.BlockSpec((128, 256), lambda i: (0, 0)),
            pl.BlockSpec((1, 256), lambda i: (0, 0)),
            pl.BlockSpec((64, 128), lambda i: (0, 0)),
            pl.BlockSpec((64, 128), lambda i: (0, 0)),
        ],
        out_specs=pl.BlockSpec((nbd, 256), lambda i: (i, 0)),
        out_shape=jax.ShapeDtypeStruct((npad, 256), f32),
        name="node_final",
    )(parts_ma, parts_mb, parts_sa, parts_sb, parts_l, z3d_d, embp,
      ne_Wc[:128], ne_Wc[128:], ne_bc.reshape(1, 128), Wm,
      bm.reshape(1, 256), wrs_aug, wrb_aug)

    return out[:n]
